# baseline (speedup 1.0000x reference)
import functools

import jax
import jax.numpy as jnp
from jax import lax
from jax.experimental import pallas as pl
from jax.experimental.pallas import tpu as pltpu
from jax.experimental.pallas import tpu_sc as plsc

B = 1024
IN_DIM = 4096
H0 = 4096
H1 = 2048
Z_DIM = 64
K_EMB = 1024

_D_PAD = 128
_KC = 256


_BN0 = 256
_BN1 = 256
_N0T = H0 // _BN0
_N1TE = H1 // _BN1


def _enc_body(x_ref, w0_ref, b0_ref, w1_ref, b1_ref, fzw_ref, fzb_ref,
              emb_ref, tab_ref, z_ref, idx_ref, h0_scr, z_scr):
    g = pl.program_id(0)

    @pl.when(g == 0)
    def _():
        tab_ref[:, :Z_DIM] = jnp.transpose(emb_ref[...])
        tab_ref[:, Z_DIM:] = jnp.zeros((K_EMB, _D_PAD - Z_DIM), jnp.float32)

    @pl.when(g < _N0T)
    def _():
        acc = lax.dot_general(x_ref[...], w0_ref[...], (((1,), (1,)), ((), ())),
                              preferred_element_type=jnp.float32)
        h0_scr[:, pl.ds(pl.multiple_of(g * _BN0, _BN0), _BN0)] = (
            jnp.maximum(acc + b0_ref[...], 0.0))

    @pl.when((g >= _N0T) & (g < _N0T + _N1TE))
    def _():
        acc = lax.dot_general(h0_scr[...], w1_ref[...], (((1,), (1,)), ((), ())),
                              preferred_element_type=jnp.float32)
        h1t = jnp.maximum(acc + b1_ref[...], 0.0)
        zp = lax.dot_general(h1t, fzw_ref[...], (((1,), (1,)), ((), ())),
                             preferred_element_type=jnp.float32)

        @pl.when(g == _N0T)
        def _():
            z_scr[...] = zp

        @pl.when(g > _N0T)
        def _():
            z_scr[...] = z_scr[...] + zp

    @pl.when(g == _N0T + _N1TE)
    def _():
        z = z_scr[...] + fzb_ref[...]
        z_ref[...] = z
        emb = emb_ref[...]
        mn = None
        am = None
        for c in range(K_EMB // _KC):
            embc = emb[:, c * _KC:(c + 1) * _KC]
            esqc = jnp.sum(embc * embc, axis=0, keepdims=True)
            crossc = lax.dot_general(z, embc, (((1,), (0,)), ((), ())),
                                     preferred_element_type=jnp.float32,
                                     precision=lax.Precision.HIGHEST)
            dc = esqc - 2.0 * crossc
            mc = jnp.min(dc, axis=1, keepdims=True)
            ac = jnp.argmin(dc, axis=1)[:, None] + c * _KC
            if c == 0:
                mn, am = mc, ac
            else:
                am = jnp.where(mc < mn, ac, am)
                mn = jnp.minimum(mc, mn)
        idx_ref[...] = am[:, 0]


def _encoder_vq(x, enc_W0, enc_b0, enc_W1, enc_b1, fz_W, fz_b, emb_W):
    last0 = _N0T - 1
    c1 = lambda g: (jnp.clip(g - _N0T, 0, _N1TE - 1), 0)
    c1b = lambda g: (0, jnp.clip(g - _N0T, 0, _N1TE - 1))
    return pl.pallas_call(
        _enc_body,
        grid=(_N0T + _N1TE + 1,),
        in_specs=[
            pl.BlockSpec((B, IN_DIM), lambda g: (0, 0)),
            pl.BlockSpec((_BN0, IN_DIM), lambda g: (jnp.minimum(g, last0), 0)),
            pl.BlockSpec((1, _BN0), lambda g: (0, jnp.minimum(g, last0))),
            pl.BlockSpec((_BN1, H0), c1),
            pl.BlockSpec((1, _BN1), c1b),
            pl.BlockSpec((Z_DIM, _BN1), c1b),
            pl.BlockSpec((1, Z_DIM), lambda g: (0, 0)),
            pl.BlockSpec((Z_DIM, K_EMB), lambda g: (0, 0)),
        ],
        out_specs=[
            pl.BlockSpec((K_EMB, _D_PAD), lambda g: (0, 0)),
            pl.BlockSpec((B, Z_DIM), lambda g: (0, 0)),
            pl.BlockSpec((B,), lambda g: (0,)),
        ],
        out_shape=[
            jax.ShapeDtypeStruct((K_EMB, _D_PAD), jnp.float32),
            jax.ShapeDtypeStruct((B, Z_DIM), jnp.float32),
            jax.ShapeDtypeStruct((B,), jnp.int32),
        ],
        scratch_shapes=[
            pltpu.VMEM((B, H0), jnp.float32),
            pltpu.VMEM((B, Z_DIM), jnp.float32),
        ],
    )(x, enc_W0, enc_b0.reshape(1, H0), enc_W1, enc_b1.reshape(1, H1),
      fz_W, fz_b.reshape(1, Z_DIM), emb_W)


_BN_D = 256
_N1T = 4096 // _BN_D
_NOT = 4096 // _BN_D


def _dec_body(idx_ref, emb_ref, w0_ref, b0_ref, w1_ref, b1_ref, wo_ref, bo_ref,
              o_ref, d0_scr, d1_scr):
    g = pl.program_id(0)

    @pl.when(g == 0)
    def _():
        am = idx_ref[...][:, None]
        emb = emb_ref[...]
        zq = None
        for c in range(K_EMB // _KC):
            embc = emb[:, c * _KC:(c + 1) * _KC]
            kc = lax.broadcasted_iota(jnp.int32, (B, _KC), 1) + c * _KC
            onehot = jnp.where(kc == am, 1.0, 0.0)
            zqc = lax.dot_general(onehot, embc, (((1,), (1,)), ((), ())),
                                  preferred_element_type=jnp.float32,
                                  precision=lax.Precision.HIGHEST)
            zq = zqc if zq is None else zq + zqc
        d0 = lax.dot_general(zq, w0_ref[...], (((1,), (1,)), ((), ())),
                             preferred_element_type=jnp.float32)
        d0_scr[...] = jnp.maximum(d0 + b0_ref[...], 0.0)

    @pl.when((g >= 1) & (g < 1 + _N1T))
    def _():
        acc = lax.dot_general(d0_scr[...], w1_ref[...], (((1,), (1,)), ((), ())),
                              preferred_element_type=jnp.float32)
        d1_scr[:, pl.ds(pl.multiple_of((g - 1) * _BN_D, _BN_D), _BN_D)] = (
            jnp.maximum(acc + b1_ref[...], 0.0))

    @pl.when(g >= 1 + _N1T)
    def _():
        acc = lax.dot_general(d1_scr[...], wo_ref[...], (((1,), (1,)), ((), ())),
                              preferred_element_type=jnp.float32)
        o_ref[...] = 1.0 / (1.0 + jnp.exp(-(acc + bo_ref[...])))


def _decoder(idx, emb_W, dec_W0, dec_b0, dec_W1, dec_b1, dec_Wout, dec_bout):
    full = lambda g: (0, 0)
    c1 = lambda g: (jnp.clip(g - 1, 0, _N1T - 1), 0)
    c1b = lambda g: (0, jnp.clip(g - 1, 0, _N1T - 1))
    co = lambda g: (jnp.clip(g - 1 - _N1T, 0, _NOT - 1), 0)
    cob = lambda g: (0, jnp.clip(g - 1 - _N1T, 0, _NOT - 1))
    return pl.pallas_call(
        _dec_body,
        grid=(1 + _N1T + _NOT,),
        in_specs=[
            pl.BlockSpec((B,), lambda g: (0,)),
            pl.BlockSpec((Z_DIM, K_EMB), full),
            pl.BlockSpec((H1, Z_DIM), full),
            pl.BlockSpec((1, H1), full),
            pl.BlockSpec((_BN_D, H1), c1),
            pl.BlockSpec((1, _BN_D), c1b),
            pl.BlockSpec((_BN_D, IN_DIM), co),
            pl.BlockSpec((1, _BN_D), cob),
        ],
        out_specs=pl.BlockSpec((B, _BN_D), cob),
        out_shape=jax.ShapeDtypeStruct((B, IN_DIM), jnp.float32),
        scratch_shapes=[
            pltpu.VMEM((B, H1), jnp.float32),
            pltpu.VMEM((B, IN_DIM), jnp.float32),
        ],
    )(idx, emb_W, dec_W0, dec_b0.reshape(1, H1), dec_W1,
      dec_b1.reshape(1, IN_DIM), dec_Wout, dec_bout.reshape(1, IN_DIM))


_SC_NC = 1
_SC_NW = 16 * _SC_NC
_SC_BPW = B // _SC_NW


def _sc_gather_body(table_hbm, idx_hbm, out_hbm, tab_sh, idx_v, rows_v, sem):
    cid = lax.axis_index("c")
    sid = lax.axis_index("s")
    wid = sid * _SC_NC + cid
    rows_per_sub = K_EMB // 16
    pltpu.sync_copy(table_hbm.at[pl.ds(sid * rows_per_sub, rows_per_sub)],
                    tab_sh.at[pl.ds(sid * rows_per_sub, rows_per_sub)])
    plsc.subcore_barrier()
    base = wid * _SC_BPW
    pltpu.sync_copy(idx_hbm.at[pl.ds(base, _SC_BPW)], idx_v)
    pltpu.async_copy(tab_sh.at[idx_v], rows_v, sem).wait()
    pltpu.sync_copy(rows_v, out_hbm.at[pl.ds(base, _SC_BPW)])


def _sc_gather(table, idx):
    mesh = plsc.VectorSubcoreMesh(core_axis_name="c", subcore_axis_name="s",
                                  num_cores=_SC_NC)
    fn = functools.partial(
        pl.kernel,
        mesh=mesh,
        out_type=jax.ShapeDtypeStruct((B, _D_PAD), jnp.float32),
        scratch_types=[
            pltpu.VMEM_SHARED((K_EMB, _D_PAD), jnp.float32),
            pltpu.VMEM((_SC_BPW,), jnp.int32),
            pltpu.VMEM((_SC_BPW, _D_PAD), jnp.float32),
            pltpu.SemaphoreType.DMA,
        ],
    )(_sc_gather_body)
    return fn(table, idx)


def kernel(x, enc_W0, enc_b0, enc_W1, enc_b1, fz_W, fz_b,
           dec_W0, dec_b0, dec_W1, dec_b1, dec_Wout, dec_bout, emb_W):
    table, z_e, idx = _encoder_vq(x, enc_W0, enc_b0, enc_W1, enc_b1,
                                  fz_W, fz_b, emb_W)
    emb_rows = _sc_gather(table, idx)[:, :Z_DIM]
    recon = _decoder(idx, emb_W, dec_W0, dec_b0, dec_W1, dec_b1,
                     dec_Wout, dec_bout)
    return (recon, z_e, emb_rows)


# speedup vs baseline: 1.0070x; 1.0070x over previous
import functools

import jax
import jax.numpy as jnp
from jax import lax
from jax.experimental import pallas as pl
from jax.experimental.pallas import tpu as pltpu
from jax.experimental.pallas import tpu_sc as plsc

B = 1024
IN_DIM = 4096
H0 = 4096
H1 = 2048
Z_DIM = 64
K_EMB = 1024

_D_PAD = 128
_KC = 512


_BN0 = 256
_BN1 = 256
_N0T = H0 // _BN0
_N1TE = H1 // _BN1


def _enc_body(x_ref, w0_ref, b0_ref, w1_ref, b1_ref, fzw_ref, fzb_ref,
              emb_ref, tab_ref, z_ref, idx_ref, h0_scr, z_scr):
    g = pl.program_id(0)

    @pl.when(g == 0)
    def _():
        tab_ref[:, :Z_DIM] = jnp.transpose(emb_ref[...])
        tab_ref[:, Z_DIM:] = jnp.zeros((K_EMB, _D_PAD - Z_DIM), jnp.float32)

    @pl.when(g < _N0T)
    def _():
        acc = lax.dot_general(x_ref[...], w0_ref[...], (((1,), (1,)), ((), ())),
                              preferred_element_type=jnp.float32)
        h0_scr[:, pl.ds(pl.multiple_of(g * _BN0, _BN0), _BN0)] = (
            jnp.maximum(acc + b0_ref[...], 0.0))

    @pl.when((g >= _N0T) & (g < _N0T + _N1TE))
    def _():
        acc = lax.dot_general(h0_scr[...], w1_ref[...], (((1,), (1,)), ((), ())),
                              preferred_element_type=jnp.float32)
        h1t = jnp.maximum(acc + b1_ref[...], 0.0)
        zp = lax.dot_general(h1t, fzw_ref[...], (((1,), (1,)), ((), ())),
                             preferred_element_type=jnp.float32)

        @pl.when(g == _N0T)
        def _():
            z_scr[...] = zp

        @pl.when(g > _N0T)
        def _():
            z_scr[...] = z_scr[...] + zp

    @pl.when(g == _N0T + _N1TE)
    def _():
        z = z_scr[...] + fzb_ref[...]
        z_ref[...] = z
        emb = emb_ref[...]
        mn = None
        am = None
        for c in range(K_EMB // _KC):
            embc = emb[:, c * _KC:(c + 1) * _KC]
            esqc = jnp.sum(embc * embc, axis=0, keepdims=True)
            crossc = lax.dot_general(z, embc, (((1,), (0,)), ((), ())),
                                     preferred_element_type=jnp.float32,
                                     precision=lax.Precision.HIGHEST)
            dc = esqc - 2.0 * crossc
            mc = jnp.min(dc, axis=1, keepdims=True)
            kc = lax.broadcasted_iota(jnp.int32, dc.shape, 1) + c * _KC
            ac = jnp.min(jnp.where(dc == mc, kc, K_EMB), axis=1, keepdims=True)
            if c == 0:
                mn, am = mc, ac
            else:
                am = jnp.where(mc < mn, ac, am)
                mn = jnp.minimum(mc, mn)
        idx_ref[...] = am[:, 0]


def _encoder_vq(x, enc_W0, enc_b0, enc_W1, enc_b1, fz_W, fz_b, emb_W):
    last0 = _N0T - 1
    c1 = lambda g: (jnp.clip(g - _N0T, 0, _N1TE - 1), 0)
    c1b = lambda g: (0, jnp.clip(g - _N0T, 0, _N1TE - 1))
    return pl.pallas_call(
        _enc_body,
        grid=(_N0T + _N1TE + 1,),
        in_specs=[
            pl.BlockSpec((B, IN_DIM), lambda g: (0, 0)),
            pl.BlockSpec((_BN0, IN_DIM), lambda g: (jnp.minimum(g, last0), 0)),
            pl.BlockSpec((1, _BN0), lambda g: (0, jnp.minimum(g, last0))),
            pl.BlockSpec((_BN1, H0), c1),
            pl.BlockSpec((1, _BN1), c1b),
            pl.BlockSpec((Z_DIM, _BN1), c1b),
            pl.BlockSpec((1, Z_DIM), lambda g: (0, 0)),
            pl.BlockSpec((Z_DIM, K_EMB), lambda g: (0, 0)),
        ],
        out_specs=[
            pl.BlockSpec((K_EMB, _D_PAD), lambda g: (0, 0)),
            pl.BlockSpec((B, Z_DIM), lambda g: (0, 0)),
            pl.BlockSpec((B,), lambda g: (0,)),
        ],
        out_shape=[
            jax.ShapeDtypeStruct((K_EMB, _D_PAD), jnp.float32),
            jax.ShapeDtypeStruct((B, Z_DIM), jnp.float32),
            jax.ShapeDtypeStruct((B,), jnp.int32),
        ],
        scratch_shapes=[
            pltpu.VMEM((B, H0), jnp.float32),
            pltpu.VMEM((B, Z_DIM), jnp.float32),
        ],
    )(x, enc_W0, enc_b0.reshape(1, H0), enc_W1, enc_b1.reshape(1, H1),
      fz_W, fz_b.reshape(1, Z_DIM), emb_W)


_BN_D = 256
_N1T = 4096 // _BN_D
_NOT = 4096 // _BN_D


def _dec_body(idx_ref, emb_ref, w0_ref, b0_ref, w1_ref, b1_ref, wo_ref, bo_ref,
              o_ref, d0_scr, d1_scr):
    g = pl.program_id(0)

    @pl.when(g == 0)
    def _():
        am = idx_ref[...][:, None]
        emb = emb_ref[...]
        zq = None
        for c in range(K_EMB // _KC):
            embc = emb[:, c * _KC:(c + 1) * _KC]
            kc = lax.broadcasted_iota(jnp.int32, (B, _KC), 1) + c * _KC
            onehot = jnp.where(kc == am, 1.0, 0.0)
            zqc = lax.dot_general(onehot, embc, (((1,), (1,)), ((), ())),
                                  preferred_element_type=jnp.float32,
                                  precision=lax.Precision.HIGHEST)
            zq = zqc if zq is None else zq + zqc
        d0 = lax.dot_general(zq, w0_ref[...], (((1,), (1,)), ((), ())),
                             preferred_element_type=jnp.float32)
        d0_scr[...] = jnp.maximum(d0 + b0_ref[...], 0.0)

    @pl.when((g >= 1) & (g < 1 + _N1T))
    def _():
        acc = lax.dot_general(d0_scr[...], w1_ref[...], (((1,), (1,)), ((), ())),
                              preferred_element_type=jnp.float32)
        d1_scr[:, pl.ds(pl.multiple_of((g - 1) * _BN_D, _BN_D), _BN_D)] = (
            jnp.maximum(acc + b1_ref[...], 0.0))

    @pl.when(g >= 1 + _N1T)
    def _():
        acc = lax.dot_general(d1_scr[...], wo_ref[...], (((1,), (1,)), ((), ())),
                              preferred_element_type=jnp.float32)
        o_ref[...] = 1.0 / (1.0 + jnp.exp(-(acc + bo_ref[...])))


def _decoder(idx, emb_W, dec_W0, dec_b0, dec_W1, dec_b1, dec_Wout, dec_bout):
    full = lambda g: (0, 0)
    c1 = lambda g: (jnp.clip(g - 1, 0, _N1T - 1), 0)
    c1b = lambda g: (0, jnp.clip(g - 1, 0, _N1T - 1))
    co = lambda g: (jnp.clip(g - 1 - _N1T, 0, _NOT - 1), 0)
    cob = lambda g: (0, jnp.clip(g - 1 - _N1T, 0, _NOT - 1))
    return pl.pallas_call(
        _dec_body,
        grid=(1 + _N1T + _NOT,),
        in_specs=[
            pl.BlockSpec((B,), lambda g: (0,)),
            pl.BlockSpec((Z_DIM, K_EMB), full),
            pl.BlockSpec((H1, Z_DIM), full),
            pl.BlockSpec((1, H1), full),
            pl.BlockSpec((_BN_D, H1), c1),
            pl.BlockSpec((1, _BN_D), c1b),
            pl.BlockSpec((_BN_D, IN_DIM), co),
            pl.BlockSpec((1, _BN_D), cob),
        ],
        out_specs=pl.BlockSpec((B, _BN_D), cob),
        out_shape=jax.ShapeDtypeStruct((B, IN_DIM), jnp.float32),
        scratch_shapes=[
            pltpu.VMEM((B, H1), jnp.float32),
            pltpu.VMEM((B, IN_DIM), jnp.float32),
        ],
    )(idx, emb_W, dec_W0, dec_b0.reshape(1, H1), dec_W1,
      dec_b1.reshape(1, IN_DIM), dec_Wout, dec_bout.reshape(1, IN_DIM))


_SC_NC = 1
_SC_NW = 16 * _SC_NC
_SC_BPW = B // _SC_NW


def _sc_gather_body(table_hbm, idx_hbm, out_hbm, tab_sh, idx_v, rows_v, sem):
    cid = lax.axis_index("c")
    sid = lax.axis_index("s")
    wid = sid * _SC_NC + cid
    rows_per_sub = K_EMB // 16
    pltpu.sync_copy(table_hbm.at[pl.ds(sid * rows_per_sub, rows_per_sub)],
                    tab_sh.at[pl.ds(sid * rows_per_sub, rows_per_sub)])
    plsc.subcore_barrier()
    base = wid * _SC_BPW
    pltpu.sync_copy(idx_hbm.at[pl.ds(base, _SC_BPW)], idx_v)
    pltpu.async_copy(tab_sh.at[idx_v], rows_v, sem).wait()
    pltpu.sync_copy(rows_v, out_hbm.at[pl.ds(base, _SC_BPW)])


def _sc_gather(table, idx):
    mesh = plsc.VectorSubcoreMesh(core_axis_name="c", subcore_axis_name="s",
                                  num_cores=_SC_NC)
    fn = functools.partial(
        pl.kernel,
        mesh=mesh,
        out_type=jax.ShapeDtypeStruct((B, _D_PAD), jnp.float32),
        scratch_types=[
            pltpu.VMEM_SHARED((K_EMB, _D_PAD), jnp.float32),
            pltpu.VMEM((_SC_BPW,), jnp.int32),
            pltpu.VMEM((_SC_BPW, _D_PAD), jnp.float32),
            pltpu.SemaphoreType.DMA,
        ],
    )(_sc_gather_body)
    return fn(table, idx)


def kernel(x, enc_W0, enc_b0, enc_W1, enc_b1, fz_W, fz_b,
           dec_W0, dec_b0, dec_W1, dec_b1, dec_Wout, dec_bout, emb_W):
    table, z_e, idx = _encoder_vq(x, enc_W0, enc_b0, enc_W1, enc_b1,
                                  fz_W, fz_b, emb_W)
    emb_rows = _sc_gather(table, idx)[:, :Z_DIM]
    recon = _decoder(idx, emb_W, dec_W0, dec_b0, dec_W1, dec_b1,
                     dec_Wout, dec_bout)
    return (recon, z_e, emb_rows)


# speedup vs baseline: 1.0331x; 1.0259x over previous
import functools

import jax
import jax.numpy as jnp
from jax import lax
from jax.experimental import pallas as pl
from jax.experimental.pallas import tpu as pltpu
from jax.experimental.pallas import tpu_sc as plsc

B = 1024
IN_DIM = 4096
H0 = 4096
H1 = 2048
Z_DIM = 64
K_EMB = 1024

_D_PAD = 128
_KC = 256


_BN0 = 256
_BN1 = 256
_N0T = H0 // _BN0
_N1TE = H1 // _BN1


def _enc_body(x_ref, w0_ref, b0_ref, w1_ref, b1_ref, fzw_ref, fzb_ref,
              emb_ref, tab_ref, z_ref, idx_ref, h0_scr, z_scr):
    g = pl.program_id(0)

    @pl.when(g == 0)
    def _():
        tab_ref[:, :Z_DIM] = jnp.transpose(emb_ref[...])
        tab_ref[:, Z_DIM:] = jnp.zeros((K_EMB, _D_PAD - Z_DIM), jnp.float32)

    @pl.when(g < _N0T)
    def _():
        acc = lax.dot_general(x_ref[...], w0_ref[...], (((1,), (1,)), ((), ())),
                              preferred_element_type=jnp.float32)
        h0_scr[:, pl.ds(pl.multiple_of(g * _BN0, _BN0), _BN0)] = (
            jnp.maximum(acc + b0_ref[...], 0.0))

    @pl.when((g >= _N0T) & (g < _N0T + _N1TE))
    def _():
        acc = lax.dot_general(h0_scr[...], w1_ref[...], (((1,), (1,)), ((), ())),
                              preferred_element_type=jnp.float32)
        h1t = jnp.maximum(acc + b1_ref[...], 0.0)
        zp = lax.dot_general(h1t, fzw_ref[...], (((1,), (1,)), ((), ())),
                             preferred_element_type=jnp.float32)

        @pl.when(g == _N0T)
        def _():
            z_scr[...] = zp

        @pl.when(g > _N0T)
        def _():
            z_scr[...] = z_scr[...] + zp

    @pl.when(g == _N0T + _N1TE)
    def _():
        z = z_scr[...] + fzb_ref[...]
        z_ref[...] = z
        emb = emb_ref[...]
        mn = None
        am = None
        for c in range(K_EMB // _KC):
            embc = emb[:, c * _KC:(c + 1) * _KC]
            esqc = jnp.sum(embc * embc, axis=0, keepdims=True)
            crossc = lax.dot_general(z, embc, (((1,), (0,)), ((), ())),
                                     preferred_element_type=jnp.float32,
                                     precision=lax.Precision.HIGHEST)
            dc = esqc - 2.0 * crossc
            mc = jnp.min(dc, axis=1, keepdims=True)
            kc = lax.broadcasted_iota(jnp.int32, dc.shape, 1) + c * _KC
            ac = jnp.min(jnp.where(dc == mc, kc, K_EMB), axis=1, keepdims=True)
            if c == 0:
                mn, am = mc, ac
            else:
                am = jnp.where(mc < mn, ac, am)
                mn = jnp.minimum(mc, mn)
        idx_ref[...] = am[:, 0]


def _encoder_vq(x, enc_W0, enc_b0, enc_W1, enc_b1, fz_W, fz_b, emb_W):
    last0 = _N0T - 1
    c1 = lambda g: (jnp.clip(g - _N0T, 0, _N1TE - 1), 0)
    c1b = lambda g: (0, jnp.clip(g - _N0T, 0, _N1TE - 1))
    return pl.pallas_call(
        _enc_body,
        grid=(_N0T + _N1TE + 1,),
        in_specs=[
            pl.BlockSpec((B, IN_DIM), lambda g: (0, 0)),
            pl.BlockSpec((_BN0, IN_DIM), lambda g: (jnp.minimum(g, last0), 0)),
            pl.BlockSpec((1, _BN0), lambda g: (0, jnp.minimum(g, last0))),
            pl.BlockSpec((_BN1, H0), c1),
            pl.BlockSpec((1, _BN1), c1b),
            pl.BlockSpec((Z_DIM, _BN1), c1b),
            pl.BlockSpec((1, Z_DIM), lambda g: (0, 0)),
            pl.BlockSpec((Z_DIM, K_EMB), lambda g: (0, 0)),
        ],
        out_specs=[
            pl.BlockSpec((K_EMB, _D_PAD), lambda g: (0, 0)),
            pl.BlockSpec((B, Z_DIM), lambda g: (0, 0)),
            pl.BlockSpec((B,), lambda g: (0,)),
        ],
        out_shape=[
            jax.ShapeDtypeStruct((K_EMB, _D_PAD), jnp.float32),
            jax.ShapeDtypeStruct((B, Z_DIM), jnp.float32),
            jax.ShapeDtypeStruct((B,), jnp.int32),
        ],
        scratch_shapes=[
            pltpu.VMEM((B, H0), jnp.float32),
            pltpu.VMEM((B, Z_DIM), jnp.float32),
        ],
    )(x, enc_W0, enc_b0.reshape(1, H0), enc_W1, enc_b1.reshape(1, H1),
      fz_W, fz_b.reshape(1, Z_DIM), emb_W)


_BN_D = 512
_N1T = 4096 // _BN_D
_NOT = 4096 // _BN_D


def _dec_body(idx_ref, emb_ref, w0_ref, b0_ref, w1_ref, b1_ref, wo_ref, bo_ref,
              o_ref, d0_scr, d1_scr):
    g = pl.program_id(0)

    @pl.when(g == 0)
    def _():
        am = idx_ref[...][:, None]
        emb = emb_ref[...]
        zq = None
        for c in range(K_EMB // _KC):
            embc = emb[:, c * _KC:(c + 1) * _KC]
            kc = lax.broadcasted_iota(jnp.int32, (B, _KC), 1) + c * _KC
            onehot = jnp.where(kc == am, 1.0, 0.0)
            zqc = lax.dot_general(onehot, embc, (((1,), (1,)), ((), ())),
                                  preferred_element_type=jnp.float32,
                                  precision=lax.Precision.HIGHEST)
            zq = zqc if zq is None else zq + zqc
        d0 = lax.dot_general(zq, w0_ref[...], (((1,), (1,)), ((), ())),
                             preferred_element_type=jnp.float32)
        d0_scr[...] = jnp.maximum(d0 + b0_ref[...], 0.0)

    @pl.when((g >= 1) & (g < 1 + _N1T))
    def _():
        acc = lax.dot_general(d0_scr[...], w1_ref[...], (((1,), (1,)), ((), ())),
                              preferred_element_type=jnp.float32)
        d1_scr[:, pl.ds(pl.multiple_of((g - 1) * _BN_D, _BN_D), _BN_D)] = (
            jnp.maximum(acc + b1_ref[...], 0.0))

    @pl.when(g >= 1 + _N1T)
    def _():
        acc = lax.dot_general(d1_scr[...], wo_ref[...], (((1,), (1,)), ((), ())),
                              preferred_element_type=jnp.float32)
        o_ref[...] = 1.0 / (1.0 + jnp.exp(-(acc + bo_ref[...])))


def _decoder(idx, emb_W, dec_W0, dec_b0, dec_W1, dec_b1, dec_Wout, dec_bout):
    full = lambda g: (0, 0)
    c1 = lambda g: (jnp.clip(g - 1, 0, _N1T - 1), 0)
    c1b = lambda g: (0, jnp.clip(g - 1, 0, _N1T - 1))
    co = lambda g: (jnp.clip(g - 1 - _N1T, 0, _NOT - 1), 0)
    cob = lambda g: (0, jnp.clip(g - 1 - _N1T, 0, _NOT - 1))
    return pl.pallas_call(
        _dec_body,
        grid=(1 + _N1T + _NOT,),
        in_specs=[
            pl.BlockSpec((B,), lambda g: (0,)),
            pl.BlockSpec((Z_DIM, K_EMB), full),
            pl.BlockSpec((H1, Z_DIM), full),
            pl.BlockSpec((1, H1), full),
            pl.BlockSpec((_BN_D, H1), c1),
            pl.BlockSpec((1, _BN_D), c1b),
            pl.BlockSpec((_BN_D, IN_DIM), co),
            pl.BlockSpec((1, _BN_D), cob),
        ],
        out_specs=pl.BlockSpec((B, _BN_D), cob),
        out_shape=jax.ShapeDtypeStruct((B, IN_DIM), jnp.float32),
        scratch_shapes=[
            pltpu.VMEM((B, H1), jnp.float32),
            pltpu.VMEM((B, IN_DIM), jnp.float32),
        ],
    )(idx, emb_W, dec_W0, dec_b0.reshape(1, H1), dec_W1,
      dec_b1.reshape(1, IN_DIM), dec_Wout, dec_bout.reshape(1, IN_DIM))


_SC_NC = 1
_SC_NW = 16 * _SC_NC
_SC_BPW = B // _SC_NW


def _sc_gather_body(table_hbm, idx_hbm, out_hbm, tab_sh, idx_v, rows_v, sem):
    cid = lax.axis_index("c")
    sid = lax.axis_index("s")
    wid = sid * _SC_NC + cid
    rows_per_sub = K_EMB // 16
    pltpu.sync_copy(table_hbm.at[pl.ds(sid * rows_per_sub, rows_per_sub)],
                    tab_sh.at[pl.ds(sid * rows_per_sub, rows_per_sub)])
    plsc.subcore_barrier()
    base = wid * _SC_BPW
    pltpu.sync_copy(idx_hbm.at[pl.ds(base, _SC_BPW)], idx_v)
    pltpu.async_copy(tab_sh.at[idx_v], rows_v, sem).wait()
    pltpu.sync_copy(rows_v, out_hbm.at[pl.ds(base, _SC_BPW)])


def _sc_gather(table, idx):
    mesh = plsc.VectorSubcoreMesh(core_axis_name="c", subcore_axis_name="s",
                                  num_cores=_SC_NC)
    fn = functools.partial(
        pl.kernel,
        mesh=mesh,
        out_type=jax.ShapeDtypeStruct((B, _D_PAD), jnp.float32),
        scratch_types=[
            pltpu.VMEM_SHARED((K_EMB, _D_PAD), jnp.float32),
            pltpu.VMEM((_SC_BPW,), jnp.int32),
            pltpu.VMEM((_SC_BPW, _D_PAD), jnp.float32),
            pltpu.SemaphoreType.DMA,
        ],
    )(_sc_gather_body)
    return fn(table, idx)


def kernel(x, enc_W0, enc_b0, enc_W1, enc_b1, fz_W, fz_b,
           dec_W0, dec_b0, dec_W1, dec_b1, dec_Wout, dec_bout, emb_W):
    table, z_e, idx = _encoder_vq(x, enc_W0, enc_b0, enc_W1, enc_b1,
                                  fz_W, fz_b, emb_W)
    emb_rows = _sc_gather(table, idx)[:, :Z_DIM]
    recon = _decoder(idx, emb_W, dec_W0, dec_b0, dec_W1, dec_b1,
                     dec_Wout, dec_bout)
    return (recon, z_e, emb_rows)
